# baseline (speedup 1.0000x reference)
import functools

import jax
import jax.numpy as jnp
from jax import lax
from jax.experimental import pallas as pl
from jax.experimental.pallas import tpu as pltpu
from jax.experimental.pallas import tpu_sc as plsc

NC = 2
NS = 16
NW = NC * NS
LANES = 16
CHUNK = 16
NBUF = 3


def _lse_body(w_ref, lse_ref):
    w = w_ref[...]
    m = jnp.max(w, axis=1, keepdims=True)
    s = jnp.sum(jnp.exp(w - m), axis=1, keepdims=True)
    lse_ref[...] = jnp.log(s) + m


def _loss_body(p_ref, o_ref, *, n_tokens):
    o_ref[...] = jnp.reshape(jnp.sum(p_ref[...]) / n_tokens, (1, 1))


def _tr_body(*refs, nt, boff, nb, aliased):
    if aliased:
        in_hbm, _, out_hbm, ibuf, obuf, isem, osem = refs
    else:
        in_hbm, out_hbm, ibuf, obuf, isem, osem = refs

    def dst(t):
        return out_hbm.at[pl.ds(t, 1), :, pl.ds(boff, nb)]

    def start_in(t, k):
        pltpu.async_copy(in_hbm.at[pl.ds(t * nb, nb), :], ibuf.at[k],
                         isem.at[k])

    start_in(0, 0)

    def body(t, carry):
        k = lax.rem(t, 2)

        @pl.when(t + 1 < nt)
        def _():
            start_in(t + 1, lax.rem(t + 1, 2))

        pltpu.make_async_copy(in_hbm.at[pl.ds(t * nb, nb), :], ibuf.at[k],
                              isem.at[k]).wait()

        @pl.when(t >= 2)
        def _():
            pltpu.make_async_copy(obuf.at[k], dst(t), osem.at[k]).wait()

        obuf[k, 0] = jnp.transpose(ibuf[k])
        pltpu.async_copy(obuf.at[k], dst(t), osem.at[k])
        return carry

    lax.fori_loop(0, nt, body, 0)
    for k in range(2):
        pltpu.make_async_copy(obuf.at[k], dst(0), osem.at[k]).wait()


def _sc_body(w_hbm, comb_hbm, meta_hbm,
             logits_hbm, part_hbm,
             meta_v, rows_v, tvals_v, lvals_v, acc_v,
             gsem, wsem, tsem, lsem,
             *, tok_per_w, n_chunks):
    wid = lax.axis_index("s") * NC + lax.axis_index("c")
    base = wid * tok_per_w
    cbase = wid * n_chunks
    acc_v[...] = jnp.zeros((LANES,), jnp.float32)

    def start_chunk(i, k):
        pltpu.sync_copy(meta_hbm.at[cbase + i], meta_v.at[k])
        pltpu.async_copy(w_hbm.at[meta_v.at[k, 0]], rows_v.at[k], gsem.at[k])
        pltpu.async_copy(comb_hbm.at[meta_v.at[k, 1]], tvals_v.at[k],
                         tsem.at[k])
        pltpu.async_copy(comb_hbm.at[meta_v.at[k, 2]], lvals_v.at[k],
                         lsem.at[k])

    def start_writeout(k):
        pltpu.async_copy(rows_v.at[k], logits_hbm.at[meta_v.at[k, 3]],
                         wsem.at[k])

    def wait_writeout(k):
        pltpu.make_async_copy(rows_v.at[k], logits_hbm.at[meta_v.at[k, 3]],
                              wsem.at[k]).wait()

    start_chunk(0, 0)
    start_chunk(1, 1)

    def body(i, carry):
        k = lax.rem(i, NBUF)

        @pl.when(i + 2 < n_chunks)
        def _():
            k2 = lax.rem(i + 2, NBUF)

            @pl.when(i >= 1)
            def _():
                wait_writeout(k2)

            start_chunk(i + 2, k2)

        pltpu.make_async_copy(w_hbm.at[meta_v.at[k, 0]], rows_v.at[k],
                              gsem.at[k]).wait()
        start_writeout(k)
        pltpu.make_async_copy(comb_hbm.at[meta_v.at[k, 1]], tvals_v.at[k],
                              tsem.at[k]).wait()
        pltpu.make_async_copy(comb_hbm.at[meta_v.at[k, 2]], lvals_v.at[k],
                              lsem.at[k]).wait()
        for j in range(CHUNK // LANES):
            sl = pl.ds(j * LANES, LANES)
            acc_v[...] = acc_v[...] + (lvals_v[k, sl] - tvals_v[k, sl])
        return carry

    lax.fori_loop(0, n_chunks, body, 0)
    for k in range(NBUF):
        wait_writeout(k)
    pltpu.sync_copy(acc_v, part_hbm.at[wid])


NSPLIT = 4


def kernel(idx, targets, W):
    b, t = idx.shape
    v, v2 = W.shape
    n = b * t
    nb_half = b // NSPLIT
    half_n = n // NSPLIT
    tok_per_w = half_n // NW
    n_chunks = tok_per_w // CHUNK

    idx_f = idx.reshape(n).astype(jnp.int32)
    tgt_f = targets.reshape(n).astype(jnp.int32)
    fidx_f = idx_f * v2 + tgt_f
    lidx_f = v * v2 + idx_f
    tok_i = jnp.arange(n, dtype=jnp.int32)
    sidx_f = (tok_i % t) * nb_half + (tok_i // t) % nb_half
    meta = jnp.stack(
        [idx_f.reshape(-1, CHUNK), fidx_f.reshape(-1, CHUNK),
         lidx_f.reshape(-1, CHUNK), sidx_f.reshape(-1, CHUNK)], axis=1)
    hc = half_n // CHUNK
    metas = [meta[hc * i: hc * (i + 1)] for i in range(NSPLIT)]

    lse = pl.pallas_call(
        _lse_body,
        out_shape=jax.ShapeDtypeStruct((v, 1), jnp.float32),
    )(W)
    comb = jnp.concatenate(
        [W.reshape(v * v2), lse.reshape(v), jnp.zeros((8,), jnp.float32)])

    mesh = plsc.VectorSubcoreMesh(core_axis_name="c", subcore_axis_name="s")
    sc = pl.kernel(
        functools.partial(_sc_body, tok_per_w=tok_per_w, n_chunks=n_chunks),
        mesh=mesh,
        out_type=[
            jax.ShapeDtypeStruct((half_n, v), jnp.float32),
            jax.ShapeDtypeStruct((NW, LANES), jnp.float32),
        ],
        scratch_types=[
            pltpu.VMEM((NBUF, 4, CHUNK), jnp.int32),
            pltpu.VMEM((NBUF, CHUNK, v), jnp.float32),
            pltpu.VMEM((NBUF, CHUNK), jnp.float32),
            pltpu.VMEM((NBUF, CHUNK), jnp.float32),
            pltpu.VMEM((LANES,), jnp.float32),
            pltpu.SemaphoreType.DMA((NBUF,)),
            pltpu.SemaphoreType.DMA((NBUF,)),
            pltpu.SemaphoreType.DMA((NBUF,)),
            pltpu.SemaphoreType.DMA((NBUF,)),
        ],
        compiler_params=pltpu.CompilerParams(use_tc_tiling_on_sc=False),
    )
    pieces = [sc(W, comb, m) for m in metas]

    loss = pl.pallas_call(
        functools.partial(_loss_body, n_tokens=float(n)),
        out_shape=jax.ShapeDtypeStruct((1, 1), jnp.float32),
    )(jnp.concatenate([p for _, p in pieces]))

    tr_scratch = [
        pltpu.VMEM((2, nb_half, v), jnp.float32),
        pltpu.VMEM((2, 1, v, nb_half), jnp.float32),
        pltpu.SemaphoreType.DMA((2,)),
        pltpu.SemaphoreType.DMA((2,)),
    ]
    hbm = pl.BlockSpec(memory_space=pltpu.MemorySpace.HBM)
    tfull = None
    for i, (lg, _) in enumerate(pieces):
        if i == 0:
            tfull = pl.pallas_call(
                functools.partial(_tr_body, nt=t, boff=0, nb=nb_half,
                                  aliased=False),
                in_specs=[hbm],
                out_specs=hbm,
                out_shape=jax.ShapeDtypeStruct((t, v, b), jnp.float32),
                scratch_shapes=tr_scratch,
            )(lg)
        else:
            tfull = pl.pallas_call(
                functools.partial(_tr_body, nt=t, boff=i * nb_half,
                                  nb=nb_half, aliased=True),
                in_specs=[hbm, hbm],
                out_specs=hbm,
                out_shape=jax.ShapeDtypeStruct((t, v, b), jnp.float32),
                input_output_aliases={1: 0},
                scratch_shapes=tr_scratch,
            )(lg, tfull)

    return (jnp.transpose(tfull, (2, 0, 1)), loss.reshape(()))


# speedup vs baseline: 1.1035x; 1.1035x over previous
import functools

import jax
import jax.numpy as jnp
from jax import lax
from jax.experimental import pallas as pl
from jax.experimental.pallas import tpu as pltpu
from jax.experimental.pallas import tpu_sc as plsc

NC = 2
NS = 16
NW = NC * NS
LANES = 16
CHUNK = 32
NBUF = 3


def _lse_body(w_ref, lse_ref):
    w = w_ref[...]
    m = jnp.max(w, axis=1, keepdims=True)
    s = jnp.sum(jnp.exp(w - m), axis=1, keepdims=True)
    lse_ref[...] = jnp.log(s) + m


def _loss_body(p_ref, o_ref, *, n_tokens):
    o_ref[...] = jnp.reshape(jnp.sum(p_ref[...]) / n_tokens, (1, 1))


def _tr_body(*refs, nt, boff, nb, aliased):
    if aliased:
        in_hbm, _, out_hbm, ibuf, obuf, isem, osem = refs
    else:
        in_hbm, out_hbm, ibuf, obuf, isem, osem = refs

    def dst(t):
        return out_hbm.at[pl.ds(t, 1), :, pl.ds(boff, nb)]

    def start_in(t, k):
        pltpu.async_copy(in_hbm.at[pl.ds(t * nb, nb), :], ibuf.at[k],
                         isem.at[k])

    start_in(0, 0)

    def body(t, carry):
        k = lax.rem(t, 2)

        @pl.when(t + 1 < nt)
        def _():
            start_in(t + 1, lax.rem(t + 1, 2))

        pltpu.make_async_copy(in_hbm.at[pl.ds(t * nb, nb), :], ibuf.at[k],
                              isem.at[k]).wait()

        @pl.when(t >= 2)
        def _():
            pltpu.make_async_copy(obuf.at[k], dst(t), osem.at[k]).wait()

        obuf[k, 0] = jnp.transpose(ibuf[k])
        pltpu.async_copy(obuf.at[k], dst(t), osem.at[k])
        return carry

    lax.fori_loop(0, nt, body, 0)
    for k in range(2):
        pltpu.make_async_copy(obuf.at[k], dst(0), osem.at[k]).wait()


def _sc_body(w_hbm, comb_hbm, meta_hbm,
             logits_hbm, part_hbm,
             meta_v, rows_v, tvals_v, lvals_v, acc_v,
             gsem, wsem, tsem, lsem,
             *, tok_per_w, n_chunks):
    wid = lax.axis_index("s") * NC + lax.axis_index("c")
    base = wid * tok_per_w
    cbase = wid * n_chunks
    acc_v[...] = jnp.zeros((LANES,), jnp.float32)

    def start_chunk(i, k):
        pltpu.sync_copy(meta_hbm.at[cbase + i], meta_v.at[k])
        pltpu.async_copy(w_hbm.at[meta_v.at[k, 0]], rows_v.at[k], gsem.at[k])
        pltpu.async_copy(comb_hbm.at[meta_v.at[k, 1]], tvals_v.at[k],
                         tsem.at[k])
        pltpu.async_copy(comb_hbm.at[meta_v.at[k, 2]], lvals_v.at[k],
                         lsem.at[k])

    def start_writeout(k):
        pltpu.async_copy(rows_v.at[k], logits_hbm.at[meta_v.at[k, 3]],
                         wsem.at[k])

    def wait_writeout(k):
        pltpu.make_async_copy(rows_v.at[k], logits_hbm.at[meta_v.at[k, 3]],
                              wsem.at[k]).wait()

    start_chunk(0, 0)
    start_chunk(1, 1)

    def body(i, carry):
        k = lax.rem(i, NBUF)

        @pl.when(i + 2 < n_chunks)
        def _():
            k2 = lax.rem(i + 2, NBUF)

            @pl.when(i >= 1)
            def _():
                wait_writeout(k2)

            start_chunk(i + 2, k2)

        pltpu.make_async_copy(w_hbm.at[meta_v.at[k, 0]], rows_v.at[k],
                              gsem.at[k]).wait()
        start_writeout(k)
        pltpu.make_async_copy(comb_hbm.at[meta_v.at[k, 1]], tvals_v.at[k],
                              tsem.at[k]).wait()
        pltpu.make_async_copy(comb_hbm.at[meta_v.at[k, 2]], lvals_v.at[k],
                              lsem.at[k]).wait()
        for j in range(CHUNK // LANES):
            sl = pl.ds(j * LANES, LANES)
            acc_v[...] = acc_v[...] + (lvals_v[k, sl] - tvals_v[k, sl])
        return carry

    lax.fori_loop(0, n_chunks, body, 0)
    for k in range(NBUF):
        wait_writeout(k)
    pltpu.sync_copy(acc_v, part_hbm.at[wid])


def kernel(idx, targets, W):
    b, t = idx.shape
    v, v2 = W.shape
    n = b * t
    nb_half = b // 2
    half_n = n // 2
    tok_per_w = half_n // NW
    n_chunks = tok_per_w // CHUNK

    idx_f = idx.reshape(n).astype(jnp.int32)
    tgt_f = targets.reshape(n).astype(jnp.int32)
    fidx_f = idx_f * v2 + tgt_f
    lidx_f = v * v2 + idx_f
    tok_i = jnp.arange(n, dtype=jnp.int32)
    sidx_f = (tok_i % t) * nb_half + (tok_i // t) % nb_half
    meta = jnp.stack(
        [idx_f.reshape(-1, CHUNK), fidx_f.reshape(-1, CHUNK),
         lidx_f.reshape(-1, CHUNK), sidx_f.reshape(-1, CHUNK)], axis=1)
    hc = half_n // CHUNK
    meta_a, meta_b = meta[:hc], meta[hc:]

    lse = pl.pallas_call(
        _lse_body,
        out_shape=jax.ShapeDtypeStruct((v, 1), jnp.float32),
    )(W)
    comb = jnp.concatenate(
        [W.reshape(v * v2), lse.reshape(v), jnp.zeros((8,), jnp.float32)])

    mesh = plsc.VectorSubcoreMesh(core_axis_name="c", subcore_axis_name="s")
    sc = pl.kernel(
        functools.partial(_sc_body, tok_per_w=tok_per_w, n_chunks=n_chunks),
        mesh=mesh,
        out_type=[
            jax.ShapeDtypeStruct((half_n, v), jnp.float32),
            jax.ShapeDtypeStruct((NW, LANES), jnp.float32),
        ],
        scratch_types=[
            pltpu.VMEM((NBUF, 4, CHUNK), jnp.int32),
            pltpu.VMEM((NBUF, CHUNK, v), jnp.float32),
            pltpu.VMEM((NBUF, CHUNK), jnp.float32),
            pltpu.VMEM((NBUF, CHUNK), jnp.float32),
            pltpu.VMEM((LANES,), jnp.float32),
            pltpu.SemaphoreType.DMA((NBUF,)),
            pltpu.SemaphoreType.DMA((NBUF,)),
            pltpu.SemaphoreType.DMA((NBUF,)),
            pltpu.SemaphoreType.DMA((NBUF,)),
        ],
        compiler_params=pltpu.CompilerParams(use_tc_tiling_on_sc=False),
    )
    logits_a, parts_a = sc(W, comb, meta_a)
    logits_b, parts_b = sc(W, comb, meta_b)

    loss = pl.pallas_call(
        functools.partial(_loss_body, n_tokens=float(n)),
        out_shape=jax.ShapeDtypeStruct((1, 1), jnp.float32),
    )(jnp.concatenate([parts_a, parts_b]))

    tr_scratch = [
        pltpu.VMEM((2, nb_half, v), jnp.float32),
        pltpu.VMEM((2, 1, v, nb_half), jnp.float32),
        pltpu.SemaphoreType.DMA((2,)),
        pltpu.SemaphoreType.DMA((2,)),
    ]
    hbm = pl.BlockSpec(memory_space=pltpu.MemorySpace.HBM)
    ta = pl.pallas_call(
        functools.partial(_tr_body, nt=t, boff=0, nb=nb_half, aliased=False),
        in_specs=[hbm],
        out_specs=hbm,
        out_shape=jax.ShapeDtypeStruct((t, v, b), jnp.float32),
        scratch_shapes=tr_scratch,
    )(logits_a)
    tfull = pl.pallas_call(
        functools.partial(_tr_body, nt=t, boff=nb_half, nb=nb_half,
                          aliased=True),
        in_specs=[hbm, hbm],
        out_specs=hbm,
        out_shape=jax.ShapeDtypeStruct((t, v, b), jnp.float32),
        input_output_aliases={1: 0},
        scratch_shapes=tr_scratch,
    )(logits_b, ta)

    return (jnp.transpose(tfull, (2, 0, 1)), loss.reshape(()))
